# baseline (speedup 1.0000x reference)
import functools
import math

import jax
import jax.numpy as jnp
import numpy as np
from jax import lax
from jax.experimental import pallas as pl
from jax.experimental.pallas import tpu as pltpu
from jax.experimental.pallas import tpu_sc as plsc

N = 10000
E = 160000
DIN = 32
DOUT = 32
NA = 4
DEDGE = 16
HID = 64

NC = 2
NS = 16
NW = NC * NS
GB = 128
E_PAD = 163840
PER_W = E_PAD // NW
CHUNKS_W = PER_W // GB
HALF = PER_W // 2
NPT = N // NS

TE = 2048
INV_SQRT_CA = 1.0 / math.sqrt(DIN * NA)
INV_SQRT_C = 1.0 / math.sqrt(DIN * 1)
C_S = math.sin(math.pi / 8.0)
C_X = math.cos(math.pi / 8.0)

_sc_mesh = plsc.VectorSubcoreMesh(core_axis_name="c", subcore_axis_name="s")
_sc_params = pltpu.CompilerParams(use_tc_tiling_on_sc=False)


def _node_pre_body(ni_ref, na_ref, wcat_ref, x_ref, s_ref):
    ni = ni_ref[...]
    na = na_ref[...]
    acc = jnp.zeros((ni.shape[0], 2 * DOUT), jnp.float32)
    for a in range(NA):
        acc = acc + na[:, a:a + 1] * jnp.dot(
            ni, wcat_ref[a], preferred_element_type=jnp.float32)
    acc = acc * INV_SQRT_CA
    x_ref[...] = acc[:, :DOUT]
    s_ref[...] = acc[:, DOUT:]


def _node_pre(ni, na, wcat):
    nt = 2000
    grid = (N // nt,)
    return pl.pallas_call(
        _node_pre_body,
        grid=grid,
        in_specs=[
            pl.BlockSpec((nt, DIN), lambda i: (i, 0)),
            pl.BlockSpec((nt, NA), lambda i: (i, 0)),
            pl.BlockSpec((NA, DIN, 2 * DOUT), lambda i: (0, 0, 0)),
        ],
        out_specs=[
            pl.BlockSpec((nt, DOUT), lambda i: (i, 0)),
            pl.BlockSpec((nt, DOUT), lambda i: (i, 0)),
        ],
        out_shape=[
            jax.ShapeDtypeStruct((N, DOUT), jnp.float32),
            jax.ShapeDtypeStruct((N, DOUT), jnp.float32),
        ],
    )(ni, na, wcat)


@functools.partial(
    pl.kernel,
    out_type=jax.ShapeDtypeStruct((E_PAD, DIN), jnp.float32),
    mesh=_sc_mesh,
    compiler_params=_sc_params,
    scratch_types=[
        pltpu.VMEM((CHUNKS_W, GB), jnp.int32),
        pltpu.VMEM((HALF, DIN), jnp.float32),
        pltpu.SemaphoreType.DMA,
    ],
)
def _gather_k(x_hbm, src_hbm, out_hbm, idx_v, rows_v, sem):
    cid = lax.axis_index("c")
    sid = lax.axis_index("s")
    wid = sid * NC + cid
    pltpu.sync_copy(src_hbm.at[pl.ds(wid * CHUNKS_W, CHUNKS_W)], idx_v)
    for half in range(2):
        cps = []
        for j in range(CHUNKS_W // 2):
            cps.append(pltpu.async_copy(
                x_hbm.at[idx_v.at[half * (CHUNKS_W // 2) + j]],
                rows_v.at[pl.ds(j * GB, GB)], sem))
        for cp in cps:
            cp.wait()
        pltpu.sync_copy(
            rows_v, out_hbm.at[pl.ds(wid * PER_W + half * HALF, HALF)])


def _edge_body(ef_ref, xs_ref, w1t_ref, b1_ref, w2t_ref, b2_ref, msg_ref):
    i = pl.program_id(0)
    eft = ef_ref[...].astype(jnp.bfloat16).T
    z = jnp.dot(w1t_ref[...], eft,
                preferred_element_type=jnp.float32) + b1_ref[...]
    h = z * jax.nn.sigmoid(z)
    full = jnp.dot(w2t_ref[...], h.astype(jnp.bfloat16),
                   preferred_element_type=jnp.float32) + b2_ref[...]
    xt = xs_ref[...].T
    acc = jnp.zeros((DOUT, TE), jnp.float32)
    for c in range(DIN):
        acc = acc + full[c * DOUT:(c + 1) * DOUT, :] * xt[c:c + 1, :]
    msgt = acc * INV_SQRT_C
    col = i * TE + lax.broadcasted_iota(jnp.int32, (DOUT, TE), 1)
    msg_ref[...] = jnp.where(col < E, msgt, 0.0).T


def _edge(ef, xs, w1t, b1c, w2t, b2c):
    grid = (E_PAD // TE,)
    return pl.pallas_call(
        _edge_body,
        grid=grid,
        in_specs=[
            pl.BlockSpec((TE, DEDGE), lambda i: (i, 0)),
            pl.BlockSpec((TE, DIN), lambda i: (i, 0)),
            pl.BlockSpec((HID, DEDGE), lambda i: (0, 0)),
            pl.BlockSpec((HID, 1), lambda i: (0, 0)),
            pl.BlockSpec((DIN * DOUT, HID), lambda i: (0, 0)),
            pl.BlockSpec((DIN * DOUT, 1), lambda i: (0, 0)),
        ],
        out_specs=pl.BlockSpec((TE, DOUT), lambda i: (i, 0)),
        out_shape=jax.ShapeDtypeStruct((E_PAD, DOUT), jnp.float32),
    )(ef, xs, w1t, b1c, w2t, b2c)


@functools.partial(
    pl.kernel,
    out_type=jax.ShapeDtypeStruct((NC, N, DOUT), jnp.float32),
    mesh=_sc_mesh,
    compiler_params=_sc_params,
    scratch_types=[
        pltpu.VMEM((CHUNKS_W, GB), jnp.int32),
        pltpu.VMEM((HALF, DOUT), jnp.float32),
        pltpu.VMEM_SHARED((N, DOUT), jnp.float32),
        pltpu.SemaphoreType.DMA,
    ],
)
def _scatter_k(msg_hbm, dst_hbm, zeros_hbm, out_hbm, idx_v, rows_v, agg_sh, sem):
    cid = lax.axis_index("c")
    sid = lax.axis_index("s")
    wid = sid * NC + cid
    pltpu.sync_copy(zeros_hbm.at[pl.ds(sid * NPT, NPT)],
                    agg_sh.at[pl.ds(sid * NPT, NPT)])
    plsc.subcore_barrier()
    pltpu.sync_copy(dst_hbm.at[pl.ds(wid * CHUNKS_W, CHUNKS_W)], idx_v)
    for half in range(2):
        pltpu.sync_copy(
            msg_hbm.at[pl.ds(wid * PER_W + half * HALF, HALF)], rows_v)
        for j in range(CHUNKS_W // 2):
            pltpu.sync_copy(
                rows_v.at[pl.ds(j * GB, GB)],
                agg_sh.at[idx_v.at[half * (CHUNKS_W // 2) + j]],
                add=True)
    plsc.subcore_barrier()
    pltpu.sync_copy(agg_sh.at[pl.ds(sid * NPT, NPT)],
                    out_hbm.at[cid].at[pl.ds(sid * NPT, NPT)])


def _final_body(a0_ref, a1_ref, na_ref, s_ref, w2_ref, out_ref):
    agg = (a0_ref[...] + a1_ref[...]) * 0.25
    na = na_ref[...]
    acc = jnp.zeros(agg.shape, jnp.float32)
    for a in range(NA):
        acc = acc + na[:, a:a + 1] * jnp.dot(
            agg, w2_ref[a], preferred_element_type=jnp.float32)
    x2 = acc * INV_SQRT_CA
    out_ref[...] = C_S * s_ref[...] + C_X * x2


def _final(a0, a1, na, s, wl2t):
    nt = 2000
    grid = (N // nt,)
    return pl.pallas_call(
        _final_body,
        grid=grid,
        in_specs=[
            pl.BlockSpec((nt, DOUT), lambda i: (i, 0)),
            pl.BlockSpec((nt, DOUT), lambda i: (i, 0)),
            pl.BlockSpec((nt, NA), lambda i: (i, 0)),
            pl.BlockSpec((nt, DOUT), lambda i: (i, 0)),
            pl.BlockSpec((NA, DIN, DOUT), lambda i: (0, 0, 0)),
        ],
        out_specs=pl.BlockSpec((nt, DOUT), lambda i: (i, 0)),
        out_shape=jax.ShapeDtypeStruct((N, DOUT), jnp.float32),
    )(a0, a1, na, s, wl2t)


def kernel(node_input, node_attr, edge_src, edge_dst, edge_features,
           W_sc, W_lin1, W_lin2, fc_w1, fc_b1, fc_w2, fc_b2):
    pad = E_PAD - E
    pad_idx = (jnp.arange(pad, dtype=jnp.int32) * 131) % N
    src_p = jnp.concatenate([edge_src, pad_idx]).reshape(E_PAD // GB, GB)
    dst_p = jnp.concatenate([edge_dst, pad_idx]).reshape(E_PAD // GB, GB)
    efp = jnp.concatenate(
        [edge_features, jnp.zeros((pad, DEDGE), jnp.float32)])

    wcat = jnp.concatenate([W_lin1, W_sc], axis=2).transpose(1, 0, 2)
    wl2t = W_lin2.transpose(1, 0, 2)
    w1t = fc_w1.T.astype(jnp.bfloat16)
    b1c = fc_b1.reshape(HID, 1)
    w2t = fc_w2.T.astype(jnp.bfloat16)
    b2c = fc_b2.reshape(DIN * DOUT, 1)

    x, s = _node_pre(node_input, node_attr, wcat)

    xg = _gather_k(x, src_p)

    msg = _edge(efp, xg, w1t, b1c, w2t, b2c)

    zeros = jnp.zeros((N, DOUT), jnp.float32)
    aggs = _scatter_k(msg, dst_p, zeros)

    return _final(aggs[0], aggs[1], node_attr, s, wl2t)


# speedup vs baseline: 1.0057x; 1.0057x over previous
import functools
import math

import jax
import jax.numpy as jnp
import numpy as np
from jax import lax
from jax.experimental import pallas as pl
from jax.experimental.pallas import tpu as pltpu
from jax.experimental.pallas import tpu_sc as plsc

N = 10000
E = 160000
DIN = 32
DOUT = 32
NA = 4
DEDGE = 16
HID = 64

NC = 2
NS = 16
NW = NC * NS
GB = 128
E_PAD = 163840
PER_W = E_PAD // NW
CHUNKS_W = PER_W // GB
HALF = PER_W // 2
NPT = N // NS

TE = 2048
INV_SQRT_CA = 1.0 / math.sqrt(DIN * NA)
INV_SQRT_C = 1.0 / math.sqrt(DIN * 1)
C_S = math.sin(math.pi / 8.0)
C_X = math.cos(math.pi / 8.0)

_sc_mesh = plsc.VectorSubcoreMesh(core_axis_name="c", subcore_axis_name="s")
_sc_params = pltpu.CompilerParams(use_tc_tiling_on_sc=False)


def _node_pre_body(ni_ref, na_ref, wcat_ref, x_ref, s_ref):
    ni = ni_ref[...]
    na = na_ref[...]
    acc = jnp.zeros((ni.shape[0], 2 * DOUT), jnp.float32)
    for a in range(NA):
        acc = acc + na[:, a:a + 1] * jnp.dot(
            ni, wcat_ref[a], preferred_element_type=jnp.float32)
    acc = acc * INV_SQRT_CA
    x_ref[...] = acc[:, :DOUT]
    s_ref[...] = acc[:, DOUT:]


def _node_pre(ni, na, wcat):
    nt = 2000
    grid = (N // nt,)
    return pl.pallas_call(
        _node_pre_body,
        grid=grid,
        in_specs=[
            pl.BlockSpec((nt, DIN), lambda i: (i, 0)),
            pl.BlockSpec((nt, NA), lambda i: (i, 0)),
            pl.BlockSpec((NA, DIN, 2 * DOUT), lambda i: (0, 0, 0)),
        ],
        out_specs=[
            pl.BlockSpec((nt, DOUT), lambda i: (i, 0)),
            pl.BlockSpec((nt, DOUT), lambda i: (i, 0)),
        ],
        out_shape=[
            jax.ShapeDtypeStruct((N, DOUT), jnp.float32),
            jax.ShapeDtypeStruct((N, DOUT), jnp.float32),
        ],
    )(ni, na, wcat)


NQ = 4
QCH = PER_W // NQ
QIDX = QCH // GB


@functools.partial(
    pl.kernel,
    out_type=jax.ShapeDtypeStruct((E_PAD, DIN), jnp.float32),
    mesh=_sc_mesh,
    compiler_params=_sc_params,
    scratch_types=[
        pltpu.VMEM((QCH,), jnp.int32),
        pltpu.VMEM((QCH,), jnp.int32),
        pltpu.VMEM((QCH,), jnp.int32),
        pltpu.VMEM((QCH,), jnp.int32),
        pltpu.VMEM((QCH, DIN), jnp.float32),
        pltpu.VMEM((QCH, DIN), jnp.float32),
        pltpu.SemaphoreType.DMA,
        pltpu.SemaphoreType.DMA,
    ],
)
def _gather_k(x_hbm, src_hbm, out_hbm, idx0, idx1, idx2, idx3,
              rows0, rows1, gsem, ssem):
    cid = lax.axis_index("c")
    sid = lax.axis_index("s")
    wid = sid * NC + cid
    bufs = (rows0, rows1)
    idxs = (idx0, idx1, idx2, idx3)
    for q in range(NQ):
        pltpu.sync_copy(src_hbm.at[pl.ds(wid * PER_W + q * QCH, QCH)], idxs[q])
    st_prev = None
    for q in range(NQ):
        cpg = pltpu.async_copy(
            x_hbm.at[idxs[q]], bufs[q % 2], gsem)
        if st_prev is not None:
            st_prev.wait()
        cpg.wait()
        st_prev = pltpu.async_copy(
            bufs[q % 2], out_hbm.at[pl.ds(wid * PER_W + q * QCH, QCH)], ssem)
    st_prev.wait()


def _edge_body(ef_ref, xs_ref, w1t_ref, b1_ref, w2t_ref, b2_ref, msg_ref):
    i = pl.program_id(0)
    eft = ef_ref[...].astype(jnp.bfloat16).T
    z = jnp.dot(w1t_ref[...], eft,
                preferred_element_type=jnp.float32) + b1_ref[...]
    h = z * jax.nn.sigmoid(z)
    full = jnp.dot(w2t_ref[...], h.astype(jnp.bfloat16),
                   preferred_element_type=jnp.float32) + b2_ref[...]
    xt = xs_ref[...].T
    acc = jnp.zeros((DOUT, TE), jnp.float32)
    for c in range(DIN):
        acc = acc + full[c * DOUT:(c + 1) * DOUT, :] * xt[c:c + 1, :]
    msgt = acc * INV_SQRT_C
    col = i * TE + lax.broadcasted_iota(jnp.int32, (DOUT, TE), 1)
    msg_ref[...] = jnp.where(col < E, msgt, 0.0).T


def _edge(ef, xs, w1t, b1c, w2t, b2c):
    grid = (E_PAD // TE,)
    return pl.pallas_call(
        _edge_body,
        grid=grid,
        in_specs=[
            pl.BlockSpec((TE, DEDGE), lambda i: (i, 0)),
            pl.BlockSpec((TE, DIN), lambda i: (i, 0)),
            pl.BlockSpec((HID, DEDGE), lambda i: (0, 0)),
            pl.BlockSpec((HID, 1), lambda i: (0, 0)),
            pl.BlockSpec((DIN * DOUT, HID), lambda i: (0, 0)),
            pl.BlockSpec((DIN * DOUT, 1), lambda i: (0, 0)),
        ],
        out_specs=pl.BlockSpec((TE, DOUT), lambda i: (i, 0)),
        out_shape=jax.ShapeDtypeStruct((E_PAD, DOUT), jnp.float32),
    )(ef, xs, w1t, b1c, w2t, b2c)


@functools.partial(
    pl.kernel,
    out_type=jax.ShapeDtypeStruct((NC, N, DOUT), jnp.float32),
    mesh=_sc_mesh,
    compiler_params=_sc_params,
    scratch_types=[
        pltpu.VMEM((QCH,), jnp.int32),
        pltpu.VMEM((QCH,), jnp.int32),
        pltpu.VMEM((QCH,), jnp.int32),
        pltpu.VMEM((QCH,), jnp.int32),
        pltpu.VMEM((QCH, DOUT), jnp.float32),
        pltpu.VMEM((QCH, DOUT), jnp.float32),
        pltpu.VMEM_SHARED((N, DOUT), jnp.float32),
        pltpu.SemaphoreType.DMA,
        pltpu.SemaphoreType.DMA,
    ],
)
def _scatter_k(msg_hbm, dst_hbm, zeros_hbm, out_hbm, idx0, idx1, idx2, idx3,
               rows0, rows1, agg_sh, lsem, asem):
    cid = lax.axis_index("c")
    sid = lax.axis_index("s")
    wid = sid * NC + cid
    bufs = (rows0, rows1)
    idxs = (idx0, idx1, idx2, idx3)
    pltpu.sync_copy(zeros_hbm.at[pl.ds(sid * NPT, NPT)],
                    agg_sh.at[pl.ds(sid * NPT, NPT)])
    for q in range(NQ):
        pltpu.sync_copy(dst_hbm.at[pl.ds(wid * PER_W + q * QCH, QCH)], idxs[q])
    plsc.subcore_barrier()
    add_prev = None
    for q in range(NQ):
        cpl = pltpu.async_copy(
            msg_hbm.at[pl.ds(wid * PER_W + q * QCH, QCH)], bufs[q % 2], lsem)
        if add_prev is not None:
            add_prev.wait()
        cpl.wait()
        add_prev = pltpu.async_copy(
            bufs[q % 2], agg_sh.at[idxs[q]],
            asem, add=True)
    add_prev.wait()
    plsc.subcore_barrier()
    pltpu.sync_copy(agg_sh.at[pl.ds(sid * NPT, NPT)],
                    out_hbm.at[cid].at[pl.ds(sid * NPT, NPT)])


def _final_body(a0_ref, a1_ref, na_ref, s_ref, w2_ref, out_ref):
    agg = (a0_ref[...] + a1_ref[...]) * 0.25
    na = na_ref[...]
    acc = jnp.zeros(agg.shape, jnp.float32)
    for a in range(NA):
        acc = acc + na[:, a:a + 1] * jnp.dot(
            agg, w2_ref[a], preferred_element_type=jnp.float32)
    x2 = acc * INV_SQRT_CA
    out_ref[...] = C_S * s_ref[...] + C_X * x2


def _final(a0, a1, na, s, wl2t):
    nt = 2000
    grid = (N // nt,)
    return pl.pallas_call(
        _final_body,
        grid=grid,
        in_specs=[
            pl.BlockSpec((nt, DOUT), lambda i: (i, 0)),
            pl.BlockSpec((nt, DOUT), lambda i: (i, 0)),
            pl.BlockSpec((nt, NA), lambda i: (i, 0)),
            pl.BlockSpec((nt, DOUT), lambda i: (i, 0)),
            pl.BlockSpec((NA, DIN, DOUT), lambda i: (0, 0, 0)),
        ],
        out_specs=pl.BlockSpec((nt, DOUT), lambda i: (i, 0)),
        out_shape=jax.ShapeDtypeStruct((N, DOUT), jnp.float32),
    )(a0, a1, na, s, wl2t)


def kernel(node_input, node_attr, edge_src, edge_dst, edge_features,
           W_sc, W_lin1, W_lin2, fc_w1, fc_b1, fc_w2, fc_b2):
    pad = E_PAD - E
    pad_idx = (jnp.arange(pad, dtype=jnp.int32) * 131) % N
    src_p = jnp.concatenate([edge_src, pad_idx])
    dst_p = jnp.concatenate([edge_dst, pad_idx])
    efp = jnp.concatenate(
        [edge_features, jnp.zeros((pad, DEDGE), jnp.float32)])

    wcat = jnp.concatenate([W_lin1, W_sc], axis=2).transpose(1, 0, 2)
    wl2t = W_lin2.transpose(1, 0, 2)
    w1t = fc_w1.T.astype(jnp.bfloat16)
    b1c = fc_b1.reshape(HID, 1)
    w2t = fc_w2.T.astype(jnp.bfloat16)
    b2c = fc_b2.reshape(DIN * DOUT, 1)

    x, s = _node_pre(node_input, node_attr, wcat)

    xg = _gather_k(x, src_p)

    msg = _edge(efp, xg, w1t, b1c, w2t, b2c)

    zeros = jnp.zeros((N, DOUT), jnp.float32)
    aggs = _scatter_k(msg, dst_p, zeros)

    return _final(aggs[0], aggs[1], node_attr, s, wl2t)


# speedup vs baseline: 1.1206x; 1.1142x over previous
import functools
import math

import jax
import jax.numpy as jnp
import numpy as np
from jax import lax
from jax.experimental import pallas as pl
from jax.experimental.pallas import tpu as pltpu
from jax.experimental.pallas import tpu_sc as plsc

N = 10000
E = 160000
DIN = 32
DOUT = 32
NA = 4
DEDGE = 16
HID = 64

NC = 2
NS = 16
NW = NC * NS
NPT = N // NS
NB = 3
NQ = 5

EHS = (81920, 78080)
TE = 1280

INV_SQRT_CA = 1.0 / math.sqrt(DIN * NA)
INV_SQRT_C = 1.0 / math.sqrt(DIN * 1)
C_S = math.sin(math.pi / 8.0)
C_X = math.cos(math.pi / 8.0)

_sc_mesh = plsc.VectorSubcoreMesh(core_axis_name="c", subcore_axis_name="s")
_sc_params = pltpu.CompilerParams(use_tc_tiling_on_sc=False)


def _node_pre_body(ni_ref, na_ref, wcat_ref, x_ref, s_ref):
    ni = ni_ref[...]
    na = na_ref[...]
    acc = jnp.zeros((ni.shape[0], 2 * DOUT), jnp.float32)
    for a in range(NA):
        acc = acc + na[:, a:a + 1] * jnp.dot(
            ni, wcat_ref[a], preferred_element_type=jnp.float32)
    acc = acc * INV_SQRT_CA
    x_ref[...] = acc[:, :DOUT]
    s_ref[...] = acc[:, DOUT:]


def _node_pre(ni, na, wcat):
    nt = 2000
    grid = (N // nt,)
    return pl.pallas_call(
        _node_pre_body,
        grid=grid,
        in_specs=[
            pl.BlockSpec((nt, DIN), lambda i: (i, 0)),
            pl.BlockSpec((nt, NA), lambda i: (i, 0)),
            pl.BlockSpec((NA, DIN, 2 * DOUT), lambda i: (0, 0, 0)),
        ],
        out_specs=[
            pl.BlockSpec((nt, DOUT), lambda i: (i, 0)),
            pl.BlockSpec((nt, DOUT), lambda i: (i, 0)),
        ],
        out_shape=[
            jax.ShapeDtypeStruct((N, DOUT), jnp.float32),
            jax.ShapeDtypeStruct((N, DOUT), jnp.float32),
        ],
    )(ni, na, wcat)


def _make_gather(eh):
    per_w = eh // NW
    qch = per_w // NQ
    assert qch % 8 == 0 and per_w % 8 == 0

    @functools.partial(
        pl.kernel,
        out_type=jax.ShapeDtypeStruct((eh, DIN), jnp.float32),
        mesh=_sc_mesh,
        compiler_params=_sc_params,
        scratch_types=[
            [pltpu.VMEM((qch,), jnp.int32) for _ in range(NQ)],
            [pltpu.VMEM((qch, DIN), jnp.float32) for _ in range(NB)],
            [pltpu.SemaphoreType.DMA for _ in range(NB)],
            [pltpu.SemaphoreType.DMA for _ in range(NB)],
        ],
    )
    def gather_k(x_hbm, src_hbm, out_hbm, idxs, bufs, gsems, ssems):
        cid = lax.axis_index("c")
        sid = lax.axis_index("s")
        wid = sid * NC + cid
        for q in range(NQ):
            pltpu.sync_copy(src_hbm.at[pl.ds(wid * per_w + q * qch, qch)],
                            idxs[q])
        gps, sts = [None] * NQ, [None] * NQ
        for q in range(NQ):
            if q >= NB:
                sts[q - NB].wait()
            gps[q] = pltpu.async_copy(x_hbm.at[idxs[q]], bufs[q % NB],
                                      gsems[q % NB])
            if q >= 1:
                gps[q - 1].wait()
                sts[q - 1] = pltpu.async_copy(
                    bufs[(q - 1) % NB],
                    out_hbm.at[pl.ds(wid * per_w + (q - 1) * qch, qch)],
                    ssems[(q - 1) % NB])
        gps[NQ - 1].wait()
        sts[NQ - 1] = pltpu.async_copy(
            bufs[(NQ - 1) % NB],
            out_hbm.at[pl.ds(wid * per_w + (NQ - 1) * qch, qch)],
            ssems[(NQ - 1) % NB])
        for q in range(max(0, NQ - NB), NQ):
            if sts[q] is not None:
                sts[q].wait()

    return gather_k


def _edge_body(eft_ref, xs_ref, w1t_ref, b1_ref, w2t_ref, b2_ref, msg_ref):
    eft = eft_ref[...].astype(jnp.bfloat16)
    z = jnp.dot(w1t_ref[...], eft,
                preferred_element_type=jnp.float32) + b1_ref[...]
    h = z * jax.nn.sigmoid(z)
    full = jnp.dot(w2t_ref[...], h.astype(jnp.bfloat16),
                   preferred_element_type=jnp.float32) + b2_ref[...]
    xt = xs_ref[...].T
    acc = jnp.zeros((DOUT, TE), jnp.float32)
    for c in range(DIN):
        acc = acc + full[c * DOUT:(c + 1) * DOUT, :] * xt[c:c + 1, :]
    msg_ref[...] = (acc * INV_SQRT_C).T


def _edge(ef, xs, w1t, b1c, w2t, b2c, tile_off, eh):
    grid = (eh // TE,)
    return pl.pallas_call(
        _edge_body,
        grid=grid,
        in_specs=[
            pl.BlockSpec((DEDGE, TE), lambda i: (0, i + tile_off)),
            pl.BlockSpec((TE, DIN), lambda i: (i, 0)),
            pl.BlockSpec((HID, DEDGE), lambda i: (0, 0)),
            pl.BlockSpec((HID, 1), lambda i: (0, 0)),
            pl.BlockSpec((DIN * DOUT, HID), lambda i: (0, 0)),
            pl.BlockSpec((DIN * DOUT, 1), lambda i: (0, 0)),
        ],
        out_specs=pl.BlockSpec((TE, DOUT), lambda i: (i, 0)),
        out_shape=jax.ShapeDtypeStruct((eh, DOUT), jnp.float32),
    )(ef, xs, w1t, b1c, w2t, b2c)


def _make_scatter(eh):
    per_w = eh // NW
    qch = per_w // NQ
    assert qch % 8 == 0 and per_w % 8 == 0

    @functools.partial(
        pl.kernel,
        out_type=jax.ShapeDtypeStruct((NC, N, DOUT), jnp.float32),
        mesh=_sc_mesh,
        compiler_params=_sc_params,
        scratch_types=[
            [pltpu.VMEM((qch,), jnp.int32) for _ in range(NQ)],
            [pltpu.VMEM((qch, DOUT), jnp.float32) for _ in range(NB)],
            pltpu.VMEM_SHARED((N, DOUT), jnp.float32),
            [pltpu.SemaphoreType.DMA for _ in range(NB)],
            [pltpu.SemaphoreType.DMA for _ in range(NB)],
        ],
    )
    def scatter_k(msg_hbm, dst_hbm, zeros_hbm, out_hbm, idxs, bufs,
                  agg_sh, lsems, asems):
        cid = lax.axis_index("c")
        sid = lax.axis_index("s")
        wid = sid * NC + cid
        pltpu.sync_copy(zeros_hbm.at[pl.ds(sid * NPT, NPT)],
                        agg_sh.at[pl.ds(sid * NPT, NPT)])
        for q in range(NQ):
            pltpu.sync_copy(dst_hbm.at[pl.ds(wid * per_w + q * qch, qch)],
                            idxs[q])
        plsc.subcore_barrier()
        lds, ads = [None] * NQ, [None] * NQ
        for q in range(NQ):
            if q >= NB:
                ads[q - NB].wait()
            lds[q] = pltpu.async_copy(
                msg_hbm.at[pl.ds(wid * per_w + q * qch, qch)], bufs[q % NB],
                lsems[q % NB])
            if q >= 1:
                lds[q - 1].wait()
                ads[q - 1] = pltpu.async_copy(
                    bufs[(q - 1) % NB], agg_sh.at[idxs[q - 1]],
                    asems[(q - 1) % NB], add=True)
        lds[NQ - 1].wait()
        ads[NQ - 1] = pltpu.async_copy(
            bufs[(NQ - 1) % NB], agg_sh.at[idxs[NQ - 1]],
            asems[(NQ - 1) % NB], add=True)
        for q in range(max(0, NQ - NB), NQ):
            if ads[q] is not None:
                ads[q].wait()
        plsc.subcore_barrier()
        pltpu.sync_copy(agg_sh.at[pl.ds(sid * NPT, NPT)],
                        out_hbm.at[cid].at[pl.ds(sid * NPT, NPT)])

    return scatter_k


_gather_ks = tuple(_make_gather(eh) for eh in EHS)
_scatter_ks = tuple(_make_scatter(eh) for eh in EHS)


def _final_body(a0_ref, a1_ref, a2_ref, a3_ref, na_ref, s_ref, w2_ref,
                out_ref):
    agg = ((a0_ref[0] + a1_ref[0]) + (a2_ref[0] + a3_ref[0])) * 0.25
    na = na_ref[...]
    acc = jnp.zeros(agg.shape, jnp.float32)
    for a in range(NA):
        acc = acc + na[:, a:a + 1] * jnp.dot(
            agg, w2_ref[a], preferred_element_type=jnp.float32)
    x2 = acc * INV_SQRT_CA
    out_ref[...] = (C_S * s_ref[...] + C_X * x2).T


def _final(a0, a1, na, s, wl2t):
    return pl.pallas_call(
        _final_body,
        grid=(1,),
        in_specs=[
            pl.BlockSpec((1, N, DOUT), lambda i: (0, 0, 0)),
            pl.BlockSpec((1, N, DOUT), lambda i: (1, 0, 0)),
            pl.BlockSpec((1, N, DOUT), lambda i: (0, 0, 0)),
            pl.BlockSpec((1, N, DOUT), lambda i: (1, 0, 0)),
            pl.BlockSpec((N, NA), lambda i: (0, 0)),
            pl.BlockSpec((N, DOUT), lambda i: (0, 0)),
            pl.BlockSpec((NA, DIN, DOUT), lambda i: (0, 0, 0)),
        ],
        out_specs=pl.BlockSpec((DOUT, N), lambda i: (0, 0)),
        out_shape=jax.ShapeDtypeStruct((DOUT, N), jnp.float32),
    )(a0, a0, a1, a1, na, s, wl2t)


def kernel(node_input, node_attr, edge_src, edge_dst, edge_features,
           W_sc, W_lin1, W_lin2, fc_w1, fc_b1, fc_w2, fc_b2):
    wcat = jnp.concatenate([W_lin1, W_sc], axis=2).transpose(1, 0, 2)
    wl2t = W_lin2.transpose(1, 0, 2)
    w1t = fc_w1.T.astype(jnp.bfloat16)
    b1c = fc_b1.reshape(HID, 1)
    w2t = fc_w2.T.astype(jnp.bfloat16)
    b2c = fc_b2.reshape(DIN * DOUT, 1)

    x, s = _node_pre(node_input, node_attr, wcat)

    zeros = jnp.zeros((N, DOUT), jnp.float32)
    eft = edge_features.T
    aggs = []
    off = 0
    for h in range(len(EHS)):
        eh = EHS[h]
        src_h = lax.slice(edge_src, (off,), (off + eh,))
        dst_h = lax.slice(edge_dst, (off,), (off + eh,))
        xg_h = _gather_ks[h](x, src_h)
        msg_h = _edge(eft, xg_h, w1t, b1c, w2t, b2c, off // TE, eh)
        aggs.append(_scatter_ks[h](msg_h, dst_h, zeros))
        off += eh

    return _final(aggs[0], aggs[1], node_attr, s, wl2t).T


# speedup vs baseline: 1.1398x; 1.0171x over previous
import functools
import math

import jax
import jax.numpy as jnp
import numpy as np
from jax import lax
from jax.experimental import pallas as pl
from jax.experimental.pallas import tpu as pltpu
from jax.experimental.pallas import tpu_sc as plsc

N = 10000
E = 160000
DIN = 32
DOUT = 32
NA = 4
DEDGE = 16
HID = 64

NC = 2
NS = 16
NW = NC * NS
NPT = N // NS
NB = 3
NQ = 5

EHS = (E,)
TE = 1280

INV_SQRT_CA = 1.0 / math.sqrt(DIN * NA)
INV_SQRT_C = 1.0 / math.sqrt(DIN * 1)
C_S = math.sin(math.pi / 8.0)
C_X = math.cos(math.pi / 8.0)

_sc_mesh = plsc.VectorSubcoreMesh(core_axis_name="c", subcore_axis_name="s")
_sc_params = pltpu.CompilerParams(use_tc_tiling_on_sc=False)


def _node_pre_body(ni_ref, na_ref, wcat_ref, x_ref, s_ref):
    ni = ni_ref[...]
    na = na_ref[...]
    acc = jnp.zeros((ni.shape[0], 2 * DOUT), jnp.float32)
    for a in range(NA):
        acc = acc + na[:, a:a + 1] * jnp.dot(
            ni, wcat_ref[a], preferred_element_type=jnp.float32)
    acc = acc * INV_SQRT_CA
    x_ref[...] = acc[:, :DOUT]
    s_ref[...] = acc[:, DOUT:]


def _node_pre(ni, na, wcat):
    nt = 2000
    grid = (N // nt,)
    return pl.pallas_call(
        _node_pre_body,
        grid=grid,
        in_specs=[
            pl.BlockSpec((nt, DIN), lambda i: (i, 0)),
            pl.BlockSpec((nt, NA), lambda i: (i, 0)),
            pl.BlockSpec((NA, DIN, 2 * DOUT), lambda i: (0, 0, 0)),
        ],
        out_specs=[
            pl.BlockSpec((nt, DOUT), lambda i: (i, 0)),
            pl.BlockSpec((nt, DOUT), lambda i: (i, 0)),
        ],
        out_shape=[
            jax.ShapeDtypeStruct((N, DOUT), jnp.float32),
            jax.ShapeDtypeStruct((N, DOUT), jnp.float32),
        ],
    )(ni, na, wcat)


def _make_gather(eh):
    per_w = eh // NW
    qch = per_w // NQ
    assert qch % 8 == 0 and per_w % 8 == 0

    @functools.partial(
        pl.kernel,
        out_type=jax.ShapeDtypeStruct((eh, DIN), jnp.float32),
        mesh=_sc_mesh,
        compiler_params=_sc_params,
        scratch_types=[
            [pltpu.VMEM((qch,), jnp.int32) for _ in range(NQ)],
            [pltpu.VMEM((qch, DIN), jnp.float32) for _ in range(NB)],
            [pltpu.SemaphoreType.DMA for _ in range(NB)],
            [pltpu.SemaphoreType.DMA for _ in range(NB)],
        ],
    )
    def gather_k(x_hbm, src_hbm, out_hbm, idxs, bufs, gsems, ssems):
        cid = lax.axis_index("c")
        sid = lax.axis_index("s")
        wid = sid * NC + cid
        for q in range(NQ):
            pltpu.sync_copy(src_hbm.at[pl.ds(wid * per_w + q * qch, qch)],
                            idxs[q])
        gps, sts = [None] * NQ, [None] * NQ
        for q in range(NQ):
            if q >= NB:
                sts[q - NB].wait()
            gps[q] = pltpu.async_copy(x_hbm.at[idxs[q]], bufs[q % NB],
                                      gsems[q % NB])
            if q >= 1:
                gps[q - 1].wait()
                sts[q - 1] = pltpu.async_copy(
                    bufs[(q - 1) % NB],
                    out_hbm.at[pl.ds(wid * per_w + (q - 1) * qch, qch)],
                    ssems[(q - 1) % NB])
        gps[NQ - 1].wait()
        sts[NQ - 1] = pltpu.async_copy(
            bufs[(NQ - 1) % NB],
            out_hbm.at[pl.ds(wid * per_w + (NQ - 1) * qch, qch)],
            ssems[(NQ - 1) % NB])
        for q in range(max(0, NQ - NB), NQ):
            if sts[q] is not None:
                sts[q].wait()

    return gather_k


def _edge_body(eft_ref, xs_ref, w1t_ref, b1_ref, w2t_ref, b2_ref, msg_ref):
    eft = eft_ref[...].astype(jnp.bfloat16)
    z = jnp.dot(w1t_ref[...], eft,
                preferred_element_type=jnp.float32) + b1_ref[...]
    h = z * jax.nn.sigmoid(z)
    full = jnp.dot(w2t_ref[...], h.astype(jnp.bfloat16),
                   preferred_element_type=jnp.float32) + b2_ref[...]
    xt = xs_ref[...].T
    acc = jnp.zeros((DOUT, TE), jnp.float32)
    for c in range(DIN):
        acc = acc + full[c * DOUT:(c + 1) * DOUT, :] * xt[c:c + 1, :]
    msg_ref[...] = (acc * INV_SQRT_C).T


def _edge(ef, xs, w1t, b1c, w2t, b2c, tile_off, eh):
    grid = (eh // TE,)
    return pl.pallas_call(
        _edge_body,
        grid=grid,
        in_specs=[
            pl.BlockSpec((DEDGE, TE), lambda i: (0, i + tile_off)),
            pl.BlockSpec((TE, DIN), lambda i: (i, 0)),
            pl.BlockSpec((HID, DEDGE), lambda i: (0, 0)),
            pl.BlockSpec((HID, 1), lambda i: (0, 0)),
            pl.BlockSpec((DIN * DOUT, HID), lambda i: (0, 0)),
            pl.BlockSpec((DIN * DOUT, 1), lambda i: (0, 0)),
        ],
        out_specs=pl.BlockSpec((TE, DOUT), lambda i: (i, 0)),
        out_shape=jax.ShapeDtypeStruct((eh, DOUT), jnp.float32),
    )(ef, xs, w1t, b1c, w2t, b2c)


def _make_scatter(eh):
    per_w = eh // NW
    qch = per_w // NQ
    assert qch % 8 == 0 and per_w % 8 == 0

    @functools.partial(
        pl.kernel,
        out_type=jax.ShapeDtypeStruct((NC, N, DOUT), jnp.float32),
        mesh=_sc_mesh,
        compiler_params=_sc_params,
        scratch_types=[
            [pltpu.VMEM((qch,), jnp.int32) for _ in range(NQ)],
            [pltpu.VMEM((qch, DOUT), jnp.float32) for _ in range(NB)],
            pltpu.VMEM_SHARED((N, DOUT), jnp.float32),
            [pltpu.SemaphoreType.DMA for _ in range(NB)],
            [pltpu.SemaphoreType.DMA for _ in range(NB)],
        ],
    )
    def scatter_k(msg_hbm, dst_hbm, zeros_hbm, out_hbm, idxs, bufs,
                  agg_sh, lsems, asems):
        cid = lax.axis_index("c")
        sid = lax.axis_index("s")
        wid = sid * NC + cid
        pltpu.sync_copy(zeros_hbm.at[pl.ds(sid * NPT, NPT)],
                        agg_sh.at[pl.ds(sid * NPT, NPT)])
        for q in range(NQ):
            pltpu.sync_copy(dst_hbm.at[pl.ds(wid * per_w + q * qch, qch)],
                            idxs[q])
        plsc.subcore_barrier()
        lds, ads = [None] * NQ, [None] * NQ
        for q in range(NQ):
            if q >= NB:
                ads[q - NB].wait()
            lds[q] = pltpu.async_copy(
                msg_hbm.at[pl.ds(wid * per_w + q * qch, qch)], bufs[q % NB],
                lsems[q % NB])
            if q >= 1:
                lds[q - 1].wait()
                ads[q - 1] = pltpu.async_copy(
                    bufs[(q - 1) % NB], agg_sh.at[idxs[q - 1]],
                    asems[(q - 1) % NB], add=True)
        lds[NQ - 1].wait()
        ads[NQ - 1] = pltpu.async_copy(
            bufs[(NQ - 1) % NB], agg_sh.at[idxs[NQ - 1]],
            asems[(NQ - 1) % NB], add=True)
        for q in range(max(0, NQ - NB), NQ):
            if ads[q] is not None:
                ads[q].wait()
        plsc.subcore_barrier()
        pltpu.sync_copy(agg_sh.at[pl.ds(sid * NPT, NPT)],
                        out_hbm.at[cid].at[pl.ds(sid * NPT, NPT)])

    return scatter_k


_gather_ks = tuple(_make_gather(eh) for eh in EHS)
_scatter_ks = tuple(_make_scatter(eh) for eh in EHS)


def _final_body(a0_ref, a1_ref, na_ref, s_ref, w2_ref, out_ref):
    agg = (a0_ref[0] + a1_ref[0]) * 0.25
    na = na_ref[...]
    acc = jnp.zeros(agg.shape, jnp.float32)
    for a in range(NA):
        acc = acc + na[:, a:a + 1] * jnp.dot(
            agg, w2_ref[a], preferred_element_type=jnp.float32)
    x2 = acc * INV_SQRT_CA
    out_ref[...] = (C_S * s_ref[...] + C_X * x2).T


def _final(a0, na, s, wl2t):
    return pl.pallas_call(
        _final_body,
        grid=(1,),
        in_specs=[
            pl.BlockSpec((1, N, DOUT), lambda i: (0, 0, 0)),
            pl.BlockSpec((1, N, DOUT), lambda i: (1, 0, 0)),
            pl.BlockSpec((N, NA), lambda i: (0, 0)),
            pl.BlockSpec((N, DOUT), lambda i: (0, 0)),
            pl.BlockSpec((NA, DIN, DOUT), lambda i: (0, 0, 0)),
        ],
        out_specs=pl.BlockSpec((DOUT, N), lambda i: (0, 0)),
        out_shape=jax.ShapeDtypeStruct((DOUT, N), jnp.float32),
    )(a0, a0, na, s, wl2t)


def kernel(node_input, node_attr, edge_src, edge_dst, edge_features,
           W_sc, W_lin1, W_lin2, fc_w1, fc_b1, fc_w2, fc_b2):
    wcat = jnp.concatenate([W_lin1, W_sc], axis=2).transpose(1, 0, 2)
    wl2t = W_lin2.transpose(1, 0, 2)
    w1t = fc_w1.T.astype(jnp.bfloat16)
    b1c = fc_b1.reshape(HID, 1)
    w2t = fc_w2.T.astype(jnp.bfloat16)
    b2c = fc_b2.reshape(DIN * DOUT, 1)

    x, s = _node_pre(node_input, node_attr, wcat)

    zeros = jnp.zeros((N, DOUT), jnp.float32)
    eft = edge_features.T
    xg = _gather_ks[0](x, edge_src)
    msg = _edge(eft, xg, w1t, b1c, w2t, b2c, 0, E)
    aggs = _scatter_ks[0](msg, edge_dst, zeros)

    return _final(aggs, node_attr, s, wl2t).T


# speedup vs baseline: 1.3103x; 1.1496x over previous
import functools
import math

import jax
import jax.numpy as jnp
import numpy as np
from jax import lax
from jax.experimental import pallas as pl
from jax.experimental.pallas import tpu as pltpu
from jax.experimental.pallas import tpu_sc as plsc

N = 10000
E = 160000
DIN = 32
DOUT = 32
NA = 4
DEDGE = 16
HID = 64

NC = 2
NS = 16
NW = NC * NS
NPT = N // NS
NB = 3
NQ = 5

EHS = (E,)
TE = 3200

INV_SQRT_CA = 1.0 / math.sqrt(DIN * NA)
INV_SQRT_C = 1.0 / math.sqrt(DIN * 1)
C_S = math.sin(math.pi / 8.0)
C_X = math.cos(math.pi / 8.0)

_sc_mesh = plsc.VectorSubcoreMesh(core_axis_name="c", subcore_axis_name="s")
_sc_params = pltpu.CompilerParams(use_tc_tiling_on_sc=False)


def _node_pre_body(ni_ref, na_ref, wcat_ref, x_ref, s_ref):
    ni = ni_ref[...]
    na = na_ref[...]
    acc = jnp.zeros((ni.shape[0], 2 * DOUT), jnp.float32)
    for a in range(NA):
        acc = acc + na[:, a:a + 1] * jnp.dot(
            ni, wcat_ref[a], preferred_element_type=jnp.float32)
    acc = acc * INV_SQRT_CA
    x_ref[...] = acc[:, :DOUT]
    s_ref[...] = acc[:, DOUT:]


def _node_pre(ni, na, wcat):
    nt = 2000
    grid = (N // nt,)
    return pl.pallas_call(
        _node_pre_body,
        grid=grid,
        in_specs=[
            pl.BlockSpec((nt, DIN), lambda i: (i, 0)),
            pl.BlockSpec((nt, NA), lambda i: (i, 0)),
            pl.BlockSpec((NA, DIN, 2 * DOUT), lambda i: (0, 0, 0)),
        ],
        out_specs=[
            pl.BlockSpec((nt, DOUT), lambda i: (i, 0)),
            pl.BlockSpec((nt, DOUT), lambda i: (i, 0)),
        ],
        out_shape=[
            jax.ShapeDtypeStruct((N, DOUT), jnp.float32),
            jax.ShapeDtypeStruct((N, DOUT), jnp.float32),
        ],
    )(ni, na, wcat)


def _make_gather(eh):
    per_w = eh // NW
    qch = per_w // NQ
    assert qch % 8 == 0 and per_w % 8 == 0

    @functools.partial(
        pl.kernel,
        out_type=jax.ShapeDtypeStruct((eh, DIN), jnp.float32),
        mesh=_sc_mesh,
        compiler_params=_sc_params,
        scratch_types=[
            [pltpu.VMEM((qch,), jnp.int32) for _ in range(NQ)],
            [pltpu.VMEM((qch, DIN), jnp.float32) for _ in range(NB)],
            [pltpu.SemaphoreType.DMA for _ in range(NB)],
            [pltpu.SemaphoreType.DMA for _ in range(NB)],
        ],
    )
    def gather_k(x_hbm, src_hbm, out_hbm, idxs, bufs, gsems, ssems):
        cid = lax.axis_index("c")
        sid = lax.axis_index("s")
        wid = sid * NC + cid
        for q in range(NQ):
            pltpu.sync_copy(src_hbm.at[pl.ds(wid * per_w + q * qch, qch)],
                            idxs[q])
        gps, sts = [None] * NQ, [None] * NQ
        for q in range(NQ):
            if q >= NB:
                sts[q - NB].wait()
            gps[q] = pltpu.async_copy(x_hbm.at[idxs[q]], bufs[q % NB],
                                      gsems[q % NB])
            if q >= 1:
                gps[q - 1].wait()
                sts[q - 1] = pltpu.async_copy(
                    bufs[(q - 1) % NB],
                    out_hbm.at[pl.ds(wid * per_w + (q - 1) * qch, qch)],
                    ssems[(q - 1) % NB])
        gps[NQ - 1].wait()
        sts[NQ - 1] = pltpu.async_copy(
            bufs[(NQ - 1) % NB],
            out_hbm.at[pl.ds(wid * per_w + (NQ - 1) * qch, qch)],
            ssems[(NQ - 1) % NB])
        for q in range(max(0, NQ - NB), NQ):
            if sts[q] is not None:
                sts[q].wait()

    return gather_k


def _edge_body(eft_ref, xs_ref, w1t_ref, b1_ref, w2t_ref, b2_ref, msg_ref):
    eft = eft_ref[...].astype(jnp.bfloat16)
    z = jnp.dot(w1t_ref[...], eft,
                preferred_element_type=jnp.float32) + b1_ref[...]
    h = z * jax.nn.sigmoid(z)
    full = jnp.dot(w2t_ref[...], h.astype(jnp.bfloat16),
                   preferred_element_type=jnp.float32) + b2_ref[...]
    xt = xs_ref[...].T
    acc = jnp.zeros((DOUT, TE), jnp.float32)
    for c in range(DIN):
        acc = acc + full[c * DOUT:(c + 1) * DOUT, :] * xt[c:c + 1, :]
    msg_ref[...] = (acc * INV_SQRT_C).T


def _edge(ef, xs, w1t, b1c, w2t, b2c, tile_off, eh):
    grid = (eh // TE,)
    return pl.pallas_call(
        _edge_body,
        grid=grid,
        in_specs=[
            pl.BlockSpec((DEDGE, TE), lambda i: (0, i + tile_off)),
            pl.BlockSpec((TE, DIN), lambda i: (i, 0)),
            pl.BlockSpec((HID, DEDGE), lambda i: (0, 0)),
            pl.BlockSpec((HID, 1), lambda i: (0, 0)),
            pl.BlockSpec((DIN * DOUT, HID), lambda i: (0, 0)),
            pl.BlockSpec((DIN * DOUT, 1), lambda i: (0, 0)),
        ],
        out_specs=pl.BlockSpec((TE, DOUT), lambda i: (i, 0)),
        out_shape=jax.ShapeDtypeStruct((eh, DOUT), jnp.float32),
    )(ef, xs, w1t, b1c, w2t, b2c)


def _make_scatter(eh):
    per_w = eh // NW
    qch = per_w // NQ
    assert qch % 8 == 0 and per_w % 8 == 0

    @functools.partial(
        pl.kernel,
        out_type=jax.ShapeDtypeStruct((NC, N, DOUT), jnp.float32),
        mesh=_sc_mesh,
        compiler_params=_sc_params,
        scratch_types=[
            [pltpu.VMEM((qch,), jnp.int32) for _ in range(NQ)],
            [pltpu.VMEM((qch, DOUT), jnp.float32) for _ in range(NB)],
            pltpu.VMEM_SHARED((N, DOUT), jnp.float32),
            [pltpu.SemaphoreType.DMA for _ in range(NB)],
            [pltpu.SemaphoreType.DMA for _ in range(NB)],
        ],
    )
    def scatter_k(msg_hbm, dst_hbm, zeros_hbm, out_hbm, idxs, bufs,
                  agg_sh, lsems, asems):
        cid = lax.axis_index("c")
        sid = lax.axis_index("s")
        wid = sid * NC + cid
        pltpu.sync_copy(zeros_hbm.at[pl.ds(sid * NPT, NPT)],
                        agg_sh.at[pl.ds(sid * NPT, NPT)])
        for q in range(NQ):
            pltpu.sync_copy(dst_hbm.at[pl.ds(wid * per_w + q * qch, qch)],
                            idxs[q])
        plsc.subcore_barrier()
        lds, ads = [None] * NQ, [None] * NQ
        for q in range(NQ):
            if q >= NB:
                ads[q - NB].wait()
            lds[q] = pltpu.async_copy(
                msg_hbm.at[pl.ds(wid * per_w + q * qch, qch)], bufs[q % NB],
                lsems[q % NB])
            if q >= 1:
                lds[q - 1].wait()
                ads[q - 1] = pltpu.async_copy(
                    bufs[(q - 1) % NB], agg_sh.at[idxs[q - 1]],
                    asems[(q - 1) % NB], add=True)
        lds[NQ - 1].wait()
        ads[NQ - 1] = pltpu.async_copy(
            bufs[(NQ - 1) % NB], agg_sh.at[idxs[NQ - 1]],
            asems[(NQ - 1) % NB], add=True)
        for q in range(max(0, NQ - NB), NQ):
            if ads[q] is not None:
                ads[q].wait()
        plsc.subcore_barrier()
        pltpu.sync_copy(agg_sh.at[pl.ds(sid * NPT, NPT)],
                        out_hbm.at[cid].at[pl.ds(sid * NPT, NPT)])

    return scatter_k


_gather_ks = tuple(_make_gather(eh) for eh in EHS)
_scatter_ks = tuple(_make_scatter(eh) for eh in EHS)


def _final_body(a0_ref, a1_ref, na_ref, s_ref, w2_ref, out_ref):
    agg = (a0_ref[0] + a1_ref[0]) * 0.25
    na = na_ref[...]
    acc = jnp.zeros(agg.shape, jnp.float32)
    for a in range(NA):
        acc = acc + na[:, a:a + 1] * jnp.dot(
            agg, w2_ref[a], preferred_element_type=jnp.float32)
    x2 = acc * INV_SQRT_CA
    out_ref[...] = (C_S * s_ref[...] + C_X * x2).T


def _final(a0, na, s, wl2t):
    return pl.pallas_call(
        _final_body,
        grid=(1,),
        in_specs=[
            pl.BlockSpec((1, N, DOUT), lambda i: (0, 0, 0)),
            pl.BlockSpec((1, N, DOUT), lambda i: (1, 0, 0)),
            pl.BlockSpec((N, NA), lambda i: (0, 0)),
            pl.BlockSpec((N, DOUT), lambda i: (0, 0)),
            pl.BlockSpec((NA, DIN, DOUT), lambda i: (0, 0, 0)),
        ],
        out_specs=pl.BlockSpec((DOUT, N), lambda i: (0, 0)),
        out_shape=jax.ShapeDtypeStruct((DOUT, N), jnp.float32),
    )(a0, a0, na, s, wl2t)


def kernel(node_input, node_attr, edge_src, edge_dst, edge_features,
           W_sc, W_lin1, W_lin2, fc_w1, fc_b1, fc_w2, fc_b2):
    wcat = jnp.concatenate([W_lin1, W_sc], axis=2).transpose(1, 0, 2)
    wl2t = W_lin2.transpose(1, 0, 2)
    w1t = fc_w1.T.astype(jnp.bfloat16)
    b1c = fc_b1.reshape(HID, 1)
    w2t = fc_w2.T.astype(jnp.bfloat16)
    b2c = fc_b2.reshape(DIN * DOUT, 1)

    x, s = _node_pre(node_input, node_attr, wcat)

    zeros = jnp.zeros((N, DOUT), jnp.float32)
    eft = edge_features.T
    xg = _gather_ks[0](x, edge_src)
    msg = _edge(eft, xg, w1t, b1c, w2t, b2c, 0, E)
    aggs = _scatter_ks[0](msg, edge_dst, zeros)

    return _final(aggs, node_attr, s, wl2t).T


# speedup vs baseline: 1.3355x; 1.0192x over previous
import functools
import math

import jax
import jax.numpy as jnp
import numpy as np
from jax import lax
from jax.experimental import pallas as pl
from jax.experimental.pallas import tpu as pltpu
from jax.experimental.pallas import tpu_sc as plsc

N = 10000
E = 160000
DIN = 32
DOUT = 32
NA = 4
DEDGE = 16
HID = 64

NC = 2
NS = 16
NW = NC * NS
NPT = N // NS
NB = 3
NQ = 5

EHS = (E,)
TE = 6400

INV_SQRT_CA = 1.0 / math.sqrt(DIN * NA)
INV_SQRT_C = 1.0 / math.sqrt(DIN * 1)
C_S = math.sin(math.pi / 8.0)
C_X = math.cos(math.pi / 8.0)

_sc_mesh = plsc.VectorSubcoreMesh(core_axis_name="c", subcore_axis_name="s")
_sc_params = pltpu.CompilerParams(use_tc_tiling_on_sc=False)


def _node_pre_body(ni_ref, na_ref, wcat_ref, x_ref, s_ref):
    ni = ni_ref[...]
    na = na_ref[...]
    acc = jnp.zeros((ni.shape[0], 2 * DOUT), jnp.float32)
    for a in range(NA):
        acc = acc + na[:, a:a + 1] * jnp.dot(
            ni, wcat_ref[a], preferred_element_type=jnp.float32)
    acc = acc * INV_SQRT_CA
    x_ref[...] = acc[:, :DOUT]
    s_ref[...] = acc[:, DOUT:]


def _node_pre(ni, na, wcat):
    nt = 2000
    grid = (N // nt,)
    return pl.pallas_call(
        _node_pre_body,
        grid=grid,
        in_specs=[
            pl.BlockSpec((nt, DIN), lambda i: (i, 0)),
            pl.BlockSpec((nt, NA), lambda i: (i, 0)),
            pl.BlockSpec((NA, DIN, 2 * DOUT), lambda i: (0, 0, 0)),
        ],
        out_specs=[
            pl.BlockSpec((nt, DOUT), lambda i: (i, 0)),
            pl.BlockSpec((nt, DOUT), lambda i: (i, 0)),
        ],
        out_shape=[
            jax.ShapeDtypeStruct((N, DOUT), jnp.float32),
            jax.ShapeDtypeStruct((N, DOUT), jnp.float32),
        ],
    )(ni, na, wcat)


def _make_gather(eh):
    per_w = eh // NW
    qch = per_w // NQ
    assert qch % 8 == 0 and per_w % 8 == 0

    @functools.partial(
        pl.kernel,
        out_type=jax.ShapeDtypeStruct((eh, DIN), jnp.float32),
        mesh=_sc_mesh,
        compiler_params=_sc_params,
        scratch_types=[
            [pltpu.VMEM((qch,), jnp.int32) for _ in range(NQ)],
            [pltpu.VMEM((qch, DIN), jnp.float32) for _ in range(NB)],
            [pltpu.SemaphoreType.DMA for _ in range(NB)],
            [pltpu.SemaphoreType.DMA for _ in range(NB)],
        ],
    )
    def gather_k(x_hbm, src_hbm, out_hbm, idxs, bufs, gsems, ssems):
        cid = lax.axis_index("c")
        sid = lax.axis_index("s")
        wid = sid * NC + cid
        for q in range(NQ):
            pltpu.sync_copy(src_hbm.at[pl.ds(wid * per_w + q * qch, qch)],
                            idxs[q])
        gps, sts = [None] * NQ, [None] * NQ
        for q in range(NQ):
            if q >= NB:
                sts[q - NB].wait()
            gps[q] = pltpu.async_copy(x_hbm.at[idxs[q]], bufs[q % NB],
                                      gsems[q % NB])
            if q >= 1:
                gps[q - 1].wait()
                sts[q - 1] = pltpu.async_copy(
                    bufs[(q - 1) % NB],
                    out_hbm.at[pl.ds(wid * per_w + (q - 1) * qch, qch)],
                    ssems[(q - 1) % NB])
        gps[NQ - 1].wait()
        sts[NQ - 1] = pltpu.async_copy(
            bufs[(NQ - 1) % NB],
            out_hbm.at[pl.ds(wid * per_w + (NQ - 1) * qch, qch)],
            ssems[(NQ - 1) % NB])
        for q in range(max(0, NQ - NB), NQ):
            if sts[q] is not None:
                sts[q].wait()

    return gather_k


def _edge_body(eft_ref, xs_ref, w1t_ref, b1_ref, w2t_ref, b2_ref, msg_ref):
    eft = eft_ref[...].astype(jnp.bfloat16)
    z = jnp.dot(w1t_ref[...], eft,
                preferred_element_type=jnp.float32) + b1_ref[...]
    h = z * jax.nn.sigmoid(z)
    full = jnp.dot(w2t_ref[...], h.astype(jnp.bfloat16),
                   preferred_element_type=jnp.float32) + b2_ref[...]
    xt = xs_ref[...].T
    acc = jnp.zeros((DOUT, TE), jnp.float32)
    for c in range(DIN):
        acc = acc + full[c * DOUT:(c + 1) * DOUT, :] * xt[c:c + 1, :]
    msg_ref[...] = (acc * INV_SQRT_C).T


def _edge(ef, xs, w1t, b1c, w2t, b2c, tile_off, eh):
    grid = (eh // TE,)
    return pl.pallas_call(
        _edge_body,
        grid=grid,
        in_specs=[
            pl.BlockSpec((DEDGE, TE), lambda i: (0, i + tile_off)),
            pl.BlockSpec((TE, DIN), lambda i: (i, 0)),
            pl.BlockSpec((HID, DEDGE), lambda i: (0, 0)),
            pl.BlockSpec((HID, 1), lambda i: (0, 0)),
            pl.BlockSpec((DIN * DOUT, HID), lambda i: (0, 0)),
            pl.BlockSpec((DIN * DOUT, 1), lambda i: (0, 0)),
        ],
        out_specs=pl.BlockSpec((TE, DOUT), lambda i: (i, 0)),
        out_shape=jax.ShapeDtypeStruct((eh, DOUT), jnp.float32),
    )(ef, xs, w1t, b1c, w2t, b2c)


def _make_scatter(eh):
    per_w = eh // NW
    qch = per_w // NQ
    assert qch % 8 == 0 and per_w % 8 == 0

    @functools.partial(
        pl.kernel,
        out_type=jax.ShapeDtypeStruct((NC, N, DOUT), jnp.float32),
        mesh=_sc_mesh,
        compiler_params=_sc_params,
        scratch_types=[
            [pltpu.VMEM((qch,), jnp.int32) for _ in range(NQ)],
            [pltpu.VMEM((qch, DOUT), jnp.float32) for _ in range(NB)],
            pltpu.VMEM_SHARED((N, DOUT), jnp.float32),
            [pltpu.SemaphoreType.DMA for _ in range(NB)],
            [pltpu.SemaphoreType.DMA for _ in range(NB)],
        ],
    )
    def scatter_k(msg_hbm, dst_hbm, zeros_hbm, out_hbm, idxs, bufs,
                  agg_sh, lsems, asems):
        cid = lax.axis_index("c")
        sid = lax.axis_index("s")
        wid = sid * NC + cid
        pltpu.sync_copy(zeros_hbm.at[pl.ds(sid * NPT, NPT)],
                        agg_sh.at[pl.ds(sid * NPT, NPT)])
        for q in range(NQ):
            pltpu.sync_copy(dst_hbm.at[pl.ds(wid * per_w + q * qch, qch)],
                            idxs[q])
        plsc.subcore_barrier()
        lds, ads = [None] * NQ, [None] * NQ
        for q in range(NQ):
            if q >= NB:
                ads[q - NB].wait()
            lds[q] = pltpu.async_copy(
                msg_hbm.at[pl.ds(wid * per_w + q * qch, qch)], bufs[q % NB],
                lsems[q % NB])
            if q >= 1:
                lds[q - 1].wait()
                ads[q - 1] = pltpu.async_copy(
                    bufs[(q - 1) % NB], agg_sh.at[idxs[q - 1]],
                    asems[(q - 1) % NB], add=True)
        lds[NQ - 1].wait()
        ads[NQ - 1] = pltpu.async_copy(
            bufs[(NQ - 1) % NB], agg_sh.at[idxs[NQ - 1]],
            asems[(NQ - 1) % NB], add=True)
        for q in range(max(0, NQ - NB), NQ):
            if ads[q] is not None:
                ads[q].wait()
        plsc.subcore_barrier()
        pltpu.sync_copy(agg_sh.at[pl.ds(sid * NPT, NPT)],
                        out_hbm.at[cid].at[pl.ds(sid * NPT, NPT)])

    return scatter_k


_gather_ks = tuple(_make_gather(eh) for eh in EHS)
_scatter_ks = tuple(_make_scatter(eh) for eh in EHS)


def _final_body(a0_ref, a1_ref, na_ref, s_ref, w2_ref, out_ref):
    agg = (a0_ref[0] + a1_ref[0]) * 0.25
    na = na_ref[...]
    acc = jnp.zeros(agg.shape, jnp.float32)
    for a in range(NA):
        acc = acc + na[:, a:a + 1] * jnp.dot(
            agg, w2_ref[a], preferred_element_type=jnp.float32)
    x2 = acc * INV_SQRT_CA
    out_ref[...] = (C_S * s_ref[...] + C_X * x2).T


def _final(a0, na, s, wl2t):
    return pl.pallas_call(
        _final_body,
        grid=(1,),
        in_specs=[
            pl.BlockSpec((1, N, DOUT), lambda i: (0, 0, 0)),
            pl.BlockSpec((1, N, DOUT), lambda i: (1, 0, 0)),
            pl.BlockSpec((N, NA), lambda i: (0, 0)),
            pl.BlockSpec((N, DOUT), lambda i: (0, 0)),
            pl.BlockSpec((NA, DIN, DOUT), lambda i: (0, 0, 0)),
        ],
        out_specs=pl.BlockSpec((DOUT, N), lambda i: (0, 0)),
        out_shape=jax.ShapeDtypeStruct((DOUT, N), jnp.float32),
    )(a0, a0, na, s, wl2t)


def kernel(node_input, node_attr, edge_src, edge_dst, edge_features,
           W_sc, W_lin1, W_lin2, fc_w1, fc_b1, fc_w2, fc_b2):
    wcat = jnp.concatenate([W_lin1, W_sc], axis=2).transpose(1, 0, 2)
    wl2t = W_lin2.transpose(1, 0, 2)
    w1t = fc_w1.T.astype(jnp.bfloat16)
    b1c = fc_b1.reshape(HID, 1)
    w2t = fc_w2.T.astype(jnp.bfloat16)
    b2c = fc_b2.reshape(DIN * DOUT, 1)

    x, s = _node_pre(node_input, node_attr, wcat)

    zeros = jnp.zeros((N, DOUT), jnp.float32)
    eft = edge_features.T
    xg = _gather_ks[0](x, edge_src)
    msg = _edge(eft, xg, w1t, b1c, w2t, b2c, 0, E)
    aggs = _scatter_ks[0](msg, edge_dst, zeros)

    return _final(aggs, node_attr, s, wl2t).T
